# Initial kernel scaffold; baseline (speedup 1.0000x reference)
import jax
import jax.numpy as jnp
from jax.experimental import pallas as pl


def kernel(query, key, value):
    raise NotImplementedError("write your pallas kernel here")


# speedup vs baseline: 1.4793x; 1.4793x over previous
import functools

import numpy as np
import jax
import jax.numpy as jnp
from jax import lax
from jax.experimental import pallas as pl
from jax.experimental.pallas import tpu as pltpu

_B, _L, _H, _D = 2, 2048, 16, 64
_U = 40
_UPAD = 48
_CH = 512
_SCALE = 1.0 / 8.0
_NEG = -1e30


@functools.lru_cache(maxsize=1)
def _count_matrix_np() -> np.ndarray:
    with jax.ensure_compile_time_eval():
        idx = np.asarray(
            jax.random.randint(jax.random.key(42), (_L, _U), 0, _L))
    cnt = np.zeros((_L, _L), np.int8)
    np.add.at(cnt, (np.arange(_L)[:, None], idx), 1)
    return cnt


def _count_matrix():
    try:
        return jnp.asarray(_count_matrix_np())
    except Exception:
        idx = jax.random.randint(jax.random.key(42), (_L, _U), 0, _L)
        cnt = jnp.zeros((_L, _L), jnp.int8)
        return cnt.at[jnp.arange(_L)[:, None], idx].add(1)


def _body(q_ref, k_ref, v_ref, cnt_ref, out_ref, mwork, ured, idxs):
    q2 = q_ref[0, 0, :, :]
    k2 = k_ref[0, 0, :, :]
    v2 = v_ref[0, 0, :, :]

    mmax = None
    msum = None
    for c in range(_L // _CH):
        kc = k2[c * _CH:(c + 1) * _CH, :]
        qk = lax.dot_general(q2, kc, (((1,), (1,)), ((), ())),
                             preferred_element_type=jnp.float32)
        cc = cnt_ref[:, c * _CH:(c + 1) * _CH].astype(jnp.float32)
        pm = jnp.max(jnp.where(cc > 0.0, qk, _NEG), axis=1, keepdims=True)
        ps = jnp.sum(qk * cc, axis=1, keepdims=True)
        mmax = pm if mmax is None else jnp.maximum(mmax, pm)
        msum = ps if msum is None else msum + ps
    mwork[...] = mmax - msum * (1.0 / _L)

    ured[...] = jnp.zeros((_UPAD, _D), jnp.float32)

    def topk_body(i, carry):
        m = mwork[...]
        mv = jnp.max(m)
        io = lax.broadcasted_iota(jnp.int32, (_L, 1), 0)
        ii = jnp.min(jnp.where(m == mv, io, _L))
        idxs[i] = ii
        mwork[pl.ds(ii, 1), :] = jnp.full((1, 1), _NEG, jnp.float32)
        ured[pl.ds(i, 1), :] = q_ref[0, 0, pl.ds(ii, 1), :]
        return carry

    lax.fori_loop(0, _U, topk_body, 0)

    sc = lax.dot_general(ured[...], k2, (((1,), (1,)), ((), ())),
                         preferred_element_type=jnp.float32) * _SCALE
    sc = sc - jnp.max(sc, axis=1, keepdims=True)
    e = jnp.exp(sc)
    attn = e / jnp.sum(e, axis=1, keepdims=True)
    upd = lax.dot_general(attn, v2, (((1,), (0,)), ((), ())),
                          preferred_element_type=jnp.float32)
    ured[...] = upd

    vs = jnp.sum(v2, axis=0, keepdims=True)
    out_ref[0, 0, :, :] = jnp.broadcast_to(vs, (_L, _D))

    def scatter_body(i, carry):
        ii = idxs[i]
        out_ref[0, 0, pl.ds(ii, 1), :] = ured[pl.ds(i, 1), :]
        return carry

    lax.fori_loop(0, _U, scatter_body, 0)


def kernel(query, key, value):
    cnt = _count_matrix()
    qt = jnp.transpose(query, (0, 2, 1, 3))
    kt = jnp.transpose(key, (0, 2, 1, 3))
    vt = jnp.transpose(value, (0, 2, 1, 3))
    bhspec = pl.BlockSpec((1, 1, _L, _D), lambda b, h: (b, h, 0, 0))
    ctx = pl.pallas_call(
        _body,
        grid=(_B, _H),
        in_specs=[
            bhspec,
            bhspec,
            bhspec,
            pl.BlockSpec((_L, _L), lambda b, h: (0, 0)),
        ],
        out_specs=bhspec,
        out_shape=jax.ShapeDtypeStruct((_B, _H, _L, _D), jnp.float32),
        scratch_shapes=[
            pltpu.VMEM((_L, 1), jnp.float32),
            pltpu.VMEM((_UPAD, _D), jnp.float32),
            pltpu.SMEM((_UPAD,), jnp.int32),
        ],
    )(qt, kt, vt, cnt)
    return jnp.transpose(ctx, (0, 2, 1, 3))


# speedup vs baseline: 2.2129x; 1.4959x over previous
import functools

import numpy as np
import jax
import jax.numpy as jnp
from jax import lax
from jax.experimental import pallas as pl
from jax.experimental.pallas import tpu as pltpu

_B, _L, _H, _D = 2, 2048, 16, 64
_U = 40
_UPAD = 48
_CH = 512
_SCALE = 1.0 / 8.0
_NEG = -1e30


@functools.lru_cache(maxsize=1)
def _count_matrix_np() -> np.ndarray:
    with jax.ensure_compile_time_eval():
        idx = np.asarray(
            jax.random.randint(jax.random.key(42), (_L, _U), 0, _L))
    cnt = np.zeros((_L, _L), np.int8)
    np.add.at(cnt, (np.arange(_L)[:, None], idx), 1)
    return np.ascontiguousarray(cnt.T)


def _count_matrix():
    try:
        return jnp.asarray(_count_matrix_np())
    except Exception:
        idx = jax.random.randint(jax.random.key(42), (_L, _U), 0, _L)
        cnt = jnp.zeros((_L, _L), jnp.int8)
        return cnt.at[jnp.arange(_L)[:, None], idx].add(1).T


def _body(q_ref, k_ref, v_ref, cnt_ref, out_ref, mwork, swork):
    q2 = q_ref[0, 0, :, :]
    k2 = k_ref[0, 0, :, :]
    v2 = v_ref[0, 0, :, :]

    mmax = None
    msum = None
    for c in range(_L // _CH):
        kc = k2[c * _CH:(c + 1) * _CH, :]
        st = lax.dot_general(kc, q2, (((1,), (1,)), ((), ())),
                             preferred_element_type=jnp.float32)
        cc = cnt_ref[c * _CH:(c + 1) * _CH, :].astype(jnp.float32)
        pm = jnp.max(jnp.where(cc > 0.0, st, _NEG), axis=0, keepdims=True)
        ps = jnp.sum(st * cc, axis=0, keepdims=True)
        mmax = pm if mmax is None else jnp.maximum(mmax, pm)
        msum = ps if msum is None else msum + ps
    mwork[...] = mmax - msum * (1.0 / _L)

    lio = lax.broadcasted_iota(jnp.int32, (1, _L), 1)
    swork[...] = jnp.full((1, _L), _L, jnp.int32)

    def topk_body(i, carry):
        m = mwork[...]
        mv = jnp.max(m, axis=(0, 1), keepdims=True)
        ii = jnp.min(jnp.where(m == mv, lio, _L), axis=(0, 1), keepdims=True)
        hit = lio == ii
        mwork[...] = jnp.where(hit, _NEG, m)
        swork[...] = jnp.where(hit, i, swork[...])
        return carry

    lax.fori_loop(0, _U, topk_body, 0)

    oh = (swork[...] == lax.broadcasted_iota(jnp.int32, (_UPAD, 1), 0)
          ).astype(jnp.float32)

    qr = lax.dot_general(oh, q2, (((1,), (0,)), ((), ())),
                         preferred_element_type=jnp.float32)
    sc = lax.dot_general(qr, k2, (((1,), (1,)), ((), ())),
                         preferred_element_type=jnp.float32) * _SCALE
    sc = sc - jnp.max(sc, axis=1, keepdims=True)
    e = jnp.exp(sc)
    attn = e / jnp.sum(e, axis=1, keepdims=True)
    upd = lax.dot_general(attn, v2, (((1,), (0,)), ((), ())),
                          preferred_element_type=jnp.float32)

    vs = jnp.sum(v2, axis=0, keepdims=True)
    delta = lax.dot_general(oh, upd - vs, (((0,), (0,)), ((), ())),
                            preferred_element_type=jnp.float32)
    out_ref[0, 0, :, :] = vs + delta


def kernel(query, key, value):
    cnt = _count_matrix()
    qt = jnp.transpose(query, (0, 2, 1, 3))
    kt = jnp.transpose(key, (0, 2, 1, 3))
    vt = jnp.transpose(value, (0, 2, 1, 3))
    bhspec = pl.BlockSpec((1, 1, _L, _D), lambda b, h: (b, h, 0, 0))
    ctx = pl.pallas_call(
        _body,
        grid=(_B, _H),
        in_specs=[
            bhspec,
            bhspec,
            bhspec,
            pl.BlockSpec((_L, _L), lambda b, h: (0, 0)),
        ],
        out_specs=bhspec,
        out_shape=jax.ShapeDtypeStruct((_B, _H, _L, _D), jnp.float32),
        scratch_shapes=[
            pltpu.VMEM((1, _L), jnp.float32),
            pltpu.VMEM((1, _L), jnp.int32),
        ],
    )(qt, kt, vt, cnt)
    return jnp.transpose(ctx, (0, 2, 1, 3))


# speedup vs baseline: 6.6947x; 3.0253x over previous
import functools

import numpy as np
import jax
import jax.numpy as jnp
from jax import lax
from jax.experimental import pallas as pl
from jax.experimental.pallas import tpu as pltpu

_B, _L, _H, _D = 2, 2048, 16, 64
_BH = _B * _H
_U = 40
_UPAD = 48
_CH = 256
_SCALE = 1.0 / 8.0
_NEG = -1e30


@functools.lru_cache(maxsize=1)
def _count_matrix_np() -> np.ndarray:
    with jax.ensure_compile_time_eval():
        idx = np.asarray(
            jax.random.randint(jax.random.key(42), (_L, _U), 0, _L))
    cnt = np.zeros((_L, _L), np.int8)
    np.add.at(cnt, (np.arange(_L)[:, None], idx), 1)
    return np.ascontiguousarray(cnt.T)


def _count_matrix():
    try:
        return jnp.asarray(_count_matrix_np())
    except Exception:
        idx = jax.random.randint(jax.random.key(42), (_L, _U), 0, _L)
        cnt = jnp.zeros((_L, _L), jnp.int8)
        return cnt.at[jnp.arange(_L)[:, None], idx].add(1).T


def _m_body(q_ref, k_ref, cnt_ref, slots_ref, mall, cntf, bias):
    i = pl.program_id(0)

    @pl.when(i == 0)
    def _expand_counts():
        for c in range(_L // _CH):
            cc = cnt_ref[c * _CH:(c + 1) * _CH, :].astype(jnp.float32)
            cntf[c * _CH:(c + 1) * _CH, :] = cc
            bias[c * _CH:(c + 1) * _CH, :] = jnp.where(cc > 0.0, 0.0, _NEG)

    @pl.when(i < _BH)
    def _stage1():
        q2 = q_ref[0, 0, :, :]
        k2 = k_ref[0, 0, :, :]
        mmax = None
        msum = None
        for c in range(_L // _CH):
            kc = k2[c * _CH:(c + 1) * _CH, :]
            st = lax.dot_general(kc, q2, (((1,), (1,)), ((), ())),
                                 preferred_element_type=jnp.float32)
            pm = jnp.max(st + bias[c * _CH:(c + 1) * _CH, :],
                         axis=0, keepdims=True)
            ps = jnp.sum(st * cntf[c * _CH:(c + 1) * _CH, :],
                         axis=0, keepdims=True)
            mmax = pm if mmax is None else jnp.maximum(mmax, pm)
            msum = ps if msum is None else msum + ps
        mall[pl.ds(i, 1), :] = mmax - msum * (1.0 / _L)

    @pl.when(i == _BH)
    def _topk():
        lio = lax.broadcasted_iota(jnp.int32, (_BH, _L), 1)

        def topk_body(t, carry):
            m, s = carry
            mv = jnp.max(m, axis=1, keepdims=True)
            ii = jnp.min(jnp.where(m == mv, lio, _L), axis=1, keepdims=True)
            hit = lio == ii
            return jnp.where(hit, _NEG, m), jnp.where(hit, t, s)

        m0 = mall[...]
        s0 = jnp.full((_BH, _L), _L, jnp.int32)
        _, s = lax.fori_loop(0, _U, topk_body, (m0, s0))
        slots_ref[...] = s


def _ctx_body(q_ref, k_ref, v_ref, slots_ref, out_ref):
    j = pl.program_id(0) * _H + pl.program_id(1)
    q2 = q_ref[0, 0, :, :]
    k2 = k_ref[0, 0, :, :]
    v2 = v_ref[0, 0, :, :]
    srow = slots_ref[pl.ds(j, 1), :]

    oh = (srow == lax.broadcasted_iota(jnp.int32, (_UPAD, 1), 0)
          ).astype(jnp.float32)

    qr = lax.dot_general(oh, q2, (((1,), (0,)), ((), ())),
                         preferred_element_type=jnp.float32)
    sc = lax.dot_general(qr, k2, (((1,), (1,)), ((), ())),
                         preferred_element_type=jnp.float32) * _SCALE
    sc = sc - jnp.max(sc, axis=1, keepdims=True)
    e = jnp.exp(sc)
    attn = e / jnp.sum(e, axis=1, keepdims=True)
    upd = lax.dot_general(attn, v2, (((1,), (0,)), ((), ())),
                          preferred_element_type=jnp.float32)

    vs = jnp.sum(v2, axis=0, keepdims=True)
    delta = lax.dot_general(oh, upd - vs, (((0,), (0,)), ((), ())),
                            preferred_element_type=jnp.float32)
    out_ref[0, 0, :, :] = vs + delta


def kernel(query, key, value):
    cnt = _count_matrix()
    qt = jnp.transpose(query, (0, 2, 1, 3))
    kt = jnp.transpose(key, (0, 2, 1, 3))
    vt = jnp.transpose(value, (0, 2, 1, 3))

    def _bh(i):
        j = jnp.minimum(i, _BH - 1)
        return (j // _H, j % _H, 0, 0)

    slots = pl.pallas_call(
        _m_body,
        grid=(_BH + 1,),
        in_specs=[
            pl.BlockSpec((1, 1, _L, _D), _bh),
            pl.BlockSpec((1, 1, _L, _D), _bh),
            pl.BlockSpec((_L, _L), lambda i: (0, 0)),
        ],
        out_specs=pl.BlockSpec((_BH, _L), lambda i: (0, 0)),
        out_shape=jax.ShapeDtypeStruct((_BH, _L), jnp.int32),
        scratch_shapes=[
            pltpu.VMEM((_BH, _L), jnp.float32),
            pltpu.VMEM((_L, _L), jnp.float32),
            pltpu.VMEM((_L, _L), jnp.float32),
        ],
    )(qt, kt, cnt)

    bhspec = pl.BlockSpec((1, 1, _L, _D), lambda b, h: (b, h, 0, 0))
    ctx = pl.pallas_call(
        _ctx_body,
        grid=(_B, _H),
        in_specs=[
            bhspec,
            bhspec,
            bhspec,
            pl.BlockSpec((_BH, _L), lambda b, h: (0, 0)),
        ],
        out_specs=bhspec,
        out_shape=jax.ShapeDtypeStruct((_B, _H, _L, _D), jnp.float32),
    )(qt, kt, vt, slots)
    return jnp.transpose(ctx, (0, 2, 1, 3))


# speedup vs baseline: 7.2086x; 1.0768x over previous
import functools

import numpy as np
import jax
import jax.numpy as jnp
from jax import lax
from jax.experimental import pallas as pl
from jax.experimental.pallas import tpu as pltpu

_B, _L, _H, _D = 2, 2048, 16, 64
_BH = _B * _H
_HD = _H * _D
_U = 40
_UPAD = 48
_CH = 256
_SCALE = 1.0 / 8.0
_NEG = -1e30


@functools.lru_cache(maxsize=1)
def _count_matrix_np() -> np.ndarray:
    with jax.ensure_compile_time_eval():
        idx = np.asarray(
            jax.random.randint(jax.random.key(42), (_L, _U), 0, _L))
    cnt = np.zeros((_L, _L), np.int8)
    np.add.at(cnt, (np.arange(_L)[:, None], idx), 1)
    return np.ascontiguousarray(cnt.T)


def _count_matrix():
    try:
        return jnp.asarray(_count_matrix_np())
    except Exception:
        idx = jax.random.randint(jax.random.key(42), (_L, _U), 0, _L)
        cnt = jnp.zeros((_L, _L), jnp.int8)
        return cnt.at[jnp.arange(_L)[:, None], idx].add(1).T


def _m_body(q_ref, k_ref, cnt_ref, slots_ref, mall, cntf, bias):
    i = pl.program_id(0)

    @pl.when(i == 0)
    def _expand_counts():
        for c in range(_L // _CH):
            cc = cnt_ref[c * _CH:(c + 1) * _CH, :].astype(jnp.float32)
            cntf[c * _CH:(c + 1) * _CH, :] = cc
            bias[c * _CH:(c + 1) * _CH, :] = jnp.where(cc > 0.0, 0.0, _NEG)

    @pl.when(i < _BH // 2)
    def _stage1():
        for ha in range(2):
            q2 = q_ref[0, :, ha * _D:(ha + 1) * _D]
            k2 = k_ref[0, :, ha * _D:(ha + 1) * _D]
            mmax = None
            msum = None
            for c in range(_L // _CH):
                kc = k2[c * _CH:(c + 1) * _CH, :]
                st = lax.dot_general(kc, q2, (((1,), (1,)), ((), ())),
                                     preferred_element_type=jnp.float32)
                pm = jnp.max(st + bias[c * _CH:(c + 1) * _CH, :],
                             axis=0, keepdims=True)
                ps = jnp.sum(st * cntf[c * _CH:(c + 1) * _CH, :],
                             axis=0, keepdims=True)
                mmax = pm if mmax is None else jnp.maximum(mmax, pm)
                msum = ps if msum is None else msum + ps
            mall[pl.ds(2 * i + ha, 1), :] = mmax - msum * (1.0 / _L)

    @pl.when(i == _BH // 2)
    def _topk():
        lio = lax.broadcasted_iota(jnp.int32, (_BH, _L), 1)

        def topk_body(t, carry):
            m, s = carry
            mv = jnp.max(m, axis=1, keepdims=True)
            ii = jnp.min(jnp.where(m == mv, lio, _L), axis=1, keepdims=True)
            hit = lio == ii
            return jnp.where(hit, _NEG, m), jnp.where(hit, t, s)

        m0 = mall[...]
        s0 = jnp.full((_BH, _L), _L, jnp.int32)
        _, s = lax.fori_loop(0, _U, topk_body, (m0, s0))
        slots_ref[...] = s


def _head_ctx(q2, k2, v2, srow):
    oh = (srow == lax.broadcasted_iota(jnp.int32, (_UPAD, 1), 0)
          ).astype(jnp.float32)
    qr = lax.dot_general(oh, q2, (((1,), (0,)), ((), ())),
                         preferred_element_type=jnp.float32)
    sc = lax.dot_general(qr, k2, (((1,), (1,)), ((), ())),
                         preferred_element_type=jnp.float32) * _SCALE
    sc = sc - jnp.max(sc, axis=1, keepdims=True)
    e = jnp.exp(sc)
    attn = e / jnp.sum(e, axis=1, keepdims=True)
    upd = lax.dot_general(attn, v2, (((1,), (0,)), ((), ())),
                          preferred_element_type=jnp.float32)
    vs = jnp.sum(v2, axis=0, keepdims=True)
    delta = lax.dot_general(oh, upd - vs, (((0,), (0,)), ((), ())),
                            preferred_element_type=jnp.float32)
    return vs + delta


def _ctx_body(q_ref, k_ref, v_ref, slots_ref, out_ref):
    i = pl.program_id(0)
    halves = []
    for ha in range(2):
        q2 = q_ref[0, :, ha * _D:(ha + 1) * _D]
        k2 = k_ref[0, :, ha * _D:(ha + 1) * _D]
        v2 = v_ref[0, :, ha * _D:(ha + 1) * _D]
        srow = slots_ref[pl.ds(2 * i + ha, 1), :]
        halves.append(_head_ctx(q2, k2, v2, srow))
    out_ref[0, :, :] = jnp.concatenate(halves, axis=1)


def kernel(query, key, value):
    cnt = _count_matrix()
    q3 = query.reshape(_B, _L, _HD)
    k3 = key.reshape(_B, _L, _HD)
    v3 = value.reshape(_B, _L, _HD)
    nhp = _H // 2

    def _pair_a(i):
        j = jnp.minimum(i, _BH // 2 - 1)
        return (j // nhp, 0, j % nhp)

    pairspec_a = pl.BlockSpec((1, _L, 2 * _D), _pair_a)
    slots = pl.pallas_call(
        _m_body,
        grid=(_BH // 2 + 1,),
        in_specs=[
            pairspec_a,
            pairspec_a,
            pl.BlockSpec((_L, _L), lambda i: (0, 0)),
        ],
        out_specs=pl.BlockSpec((_BH, _L), lambda i: (0, 0)),
        out_shape=jax.ShapeDtypeStruct((_BH, _L), jnp.int32),
        scratch_shapes=[
            pltpu.VMEM((_BH, _L), jnp.float32),
            pltpu.VMEM((_L, _L), jnp.float32),
            pltpu.VMEM((_L, _L), jnp.float32),
        ],
    )(q3, k3, cnt)

    pairspec = pl.BlockSpec((1, _L, 2 * _D),
                            lambda i: (i // nhp, 0, i % nhp))
    ctx = pl.pallas_call(
        _ctx_body,
        grid=(_BH // 2,),
        in_specs=[
            pairspec,
            pairspec,
            pairspec,
            pl.BlockSpec((_BH, _L), lambda i: (0, 0)),
        ],
        out_specs=pairspec,
        out_shape=jax.ShapeDtypeStruct((_B, _L, _HD), jnp.float32),
    )(q3, k3, v3, slots)
    return ctx.reshape(_B, _L, _H, _D)


# speedup vs baseline: 7.2359x; 1.0038x over previous
import functools

import numpy as np
import jax
import jax.numpy as jnp
from jax import lax
from jax.experimental import pallas as pl
from jax.experimental.pallas import tpu as pltpu

_B, _L, _H, _D = 2, 2048, 16, 64
_BH = _B * _H
_HD = _H * _D
_NP = _BH // 2
_U = 40
_UPAD = 48
_CH = 256
_SCALE = 1.0 / 8.0
_NEG = -1e30


@functools.lru_cache(maxsize=1)
def _count_matrix_np() -> np.ndarray:
    with jax.ensure_compile_time_eval():
        idx = np.asarray(
            jax.random.randint(jax.random.key(42), (_L, _U), 0, _L))
    cnt = np.zeros((_L, _L), np.int8)
    np.add.at(cnt, (np.arange(_L)[:, None], idx), 1)
    return np.ascontiguousarray(cnt.T)


def _count_matrix():
    try:
        return jnp.asarray(_count_matrix_np())
    except Exception:
        idx = jax.random.randint(jax.random.key(42), (_L, _U), 0, _L)
        cnt = jnp.zeros((_L, _L), jnp.int8)
        return cnt.at[jnp.arange(_L)[:, None], idx].add(1).T


def _head_ctx(q2, k2, v2, srow):
    oh = (srow == lax.broadcasted_iota(jnp.int32, (_UPAD, 1), 0)
          ).astype(jnp.float32)
    qr = lax.dot_general(oh, q2, (((1,), (0,)), ((), ())),
                         preferred_element_type=jnp.float32)
    sc = lax.dot_general(qr, k2, (((1,), (1,)), ((), ())),
                         preferred_element_type=jnp.float32) * _SCALE
    sc = sc - jnp.max(sc, axis=1, keepdims=True)
    e = jnp.exp(sc)
    attn = e / jnp.sum(e, axis=1, keepdims=True)
    upd = lax.dot_general(attn, v2, (((1,), (0,)), ((), ())),
                          preferred_element_type=jnp.float32)
    vs = jnp.sum(v2, axis=0, keepdims=True)
    delta = lax.dot_general(oh, upd - vs, (((0,), (0,)), ((), ())),
                            preferred_element_type=jnp.float32)
    return vs + delta


def _body(q_ref, k_ref, v_ref, cnt_ref, out_ref, mall, sall, cntf, bias):
    i = pl.program_id(0)

    @pl.when(i == 0)
    def _expand_counts():
        for c in range(_L // _CH):
            cc = cnt_ref[c * _CH:(c + 1) * _CH, :].astype(jnp.float32)
            cntf[c * _CH:(c + 1) * _CH, :] = cc
            bias[c * _CH:(c + 1) * _CH, :] = jnp.where(cc > 0.0, 0.0, _NEG)

    @pl.when(i < _NP)
    def _stage1():
        for ha in range(2):
            q2 = q_ref[0, :, ha * _D:(ha + 1) * _D]
            k2 = k_ref[0, :, ha * _D:(ha + 1) * _D]
            mmax = None
            msum = None
            for c in range(_L // _CH):
                kc = k2[c * _CH:(c + 1) * _CH, :]
                st = lax.dot_general(kc, q2, (((1,), (1,)), ((), ())),
                                     preferred_element_type=jnp.float32)
                pm = jnp.max(st + bias[c * _CH:(c + 1) * _CH, :],
                             axis=0, keepdims=True)
                ps = jnp.sum(st * cntf[c * _CH:(c + 1) * _CH, :],
                             axis=0, keepdims=True)
                mmax = pm if mmax is None else jnp.maximum(mmax, pm)
                msum = ps if msum is None else msum + ps
            mall[pl.ds(2 * i + ha, 1), :] = mmax - msum * (1.0 / _L)

    @pl.when(i == _NP)
    def _topk():
        lio = lax.broadcasted_iota(jnp.int32, (_BH, _L), 1)

        def topk_body(t, carry):
            m, s = carry
            mv = jnp.max(m, axis=1, keepdims=True)
            ii = jnp.min(jnp.where(m == mv, lio, _L), axis=1, keepdims=True)
            hit = lio == ii
            return jnp.where(hit, _NEG, m), jnp.where(hit, t, s)

        m0 = mall[...]
        s0 = jnp.full((_BH, _L), _L, jnp.int32)
        _, s = lax.fori_loop(0, _U, topk_body, (m0, s0))
        sall[...] = s

    @pl.when(i > _NP)
    def _stage2():
        p = i - (_NP + 1)
        halves = []
        for ha in range(2):
            q2 = q_ref[0, :, ha * _D:(ha + 1) * _D]
            k2 = k_ref[0, :, ha * _D:(ha + 1) * _D]
            v2 = v_ref[0, :, ha * _D:(ha + 1) * _D]
            srow = sall[pl.ds(2 * p + ha, 1), :]
            halves.append(_head_ctx(q2, k2, v2, srow))
        out_ref[0, :, :] = jnp.concatenate(halves, axis=1)


def kernel(query, key, value):
    cnt = _count_matrix()
    q3 = query.reshape(_B, _L, _HD)
    k3 = key.reshape(_B, _L, _HD)
    v3 = value.reshape(_B, _L, _HD)
    nhp = _H // 2

    def _pair_qk(i):
        j = jnp.where(i < _NP, i, jnp.maximum(i - (_NP + 1), 0))
        return (j // nhp, 0, j % nhp)

    def _pair_out(i):
        j = jnp.maximum(i - (_NP + 1), 0)
        return (j // nhp, 0, j % nhp)

    pair_qk = pl.BlockSpec((1, _L, 2 * _D), _pair_qk)
    pair_out = pl.BlockSpec((1, _L, 2 * _D), _pair_out)
    ctx = pl.pallas_call(
        _body,
        grid=(2 * _NP + 1,),
        in_specs=[
            pair_qk,
            pair_qk,
            pair_out,
            pl.BlockSpec((_L, _L), lambda i: (0, 0)),
        ],
        out_specs=pair_out,
        out_shape=jax.ShapeDtypeStruct((_B, _L, _HD), jnp.float32),
        scratch_shapes=[
            pltpu.VMEM((_BH, _L), jnp.float32),
            pltpu.VMEM((_BH, _L), jnp.int32),
            pltpu.VMEM((_L, _L), jnp.float32),
            pltpu.VMEM((_L, _L), jnp.float32),
        ],
    )(q3, k3, v3, cnt)
    return ctx.reshape(_B, _L, _H, _D)


# speedup vs baseline: 7.4098x; 1.0240x over previous
import functools

import numpy as np
import jax
import jax.numpy as jnp
from jax import lax
from jax.experimental import pallas as pl
from jax.experimental.pallas import tpu as pltpu

_B, _L, _H, _D = 2, 2048, 16, 64
_BH = _B * _H
_HD = _H * _D
_NP = _BH // 2
_U = 40
_UPAD = 48
_CH = 512
_SCALE = 1.0 / 8.0
_NEG = -1e30


@functools.lru_cache(maxsize=1)
def _count_matrix_np() -> np.ndarray:
    with jax.ensure_compile_time_eval():
        idx = np.asarray(
            jax.random.randint(jax.random.key(42), (_L, _U), 0, _L))
    cnt = np.zeros((_L, _L), np.int8)
    np.add.at(cnt, (np.arange(_L)[:, None], idx), 1)
    return np.ascontiguousarray(cnt.T)


def _count_matrix():
    try:
        return jnp.asarray(_count_matrix_np())
    except Exception:
        idx = jax.random.randint(jax.random.key(42), (_L, _U), 0, _L)
        cnt = jnp.zeros((_L, _L), jnp.int8)
        return cnt.at[jnp.arange(_L)[:, None], idx].add(1).T


def _head_ctx(q2, k2, v2, srow):
    oh = (srow == lax.broadcasted_iota(jnp.int32, (_UPAD, 1), 0)
          ).astype(jnp.float32)
    qr = lax.dot_general(oh, q2, (((1,), (0,)), ((), ())),
                         preferred_element_type=jnp.float32)
    sc = lax.dot_general(qr, k2, (((1,), (1,)), ((), ())),
                         preferred_element_type=jnp.float32) * _SCALE
    sc = sc - jnp.max(sc, axis=1, keepdims=True)
    e = jnp.exp(sc)
    attn = e / jnp.sum(e, axis=1, keepdims=True)
    upd = lax.dot_general(attn, v2, (((1,), (0,)), ((), ())),
                          preferred_element_type=jnp.float32)
    vs = jnp.sum(v2, axis=0, keepdims=True)
    delta = lax.dot_general(oh, upd - vs, (((0,), (0,)), ((), ())),
                            preferred_element_type=jnp.float32)
    return vs + delta


def _body(q_ref, k_ref, v_ref, cnt_ref, out_ref, mall, sall, cntf, bias):
    i = pl.program_id(0)

    @pl.when(i == 0)
    def _expand_counts():
        for c in range(_L // _CH):
            cc = cnt_ref[c * _CH:(c + 1) * _CH, :].astype(jnp.float32)
            cntf[c * _CH:(c + 1) * _CH, :] = cc
            bias[c * _CH:(c + 1) * _CH, :] = jnp.where(cc > 0.0, 0.0, _NEG)

    @pl.when(i < _NP)
    def _stage1():
        for ha in range(2):
            q2 = q_ref[0, :, ha * _D:(ha + 1) * _D]
            k2 = k_ref[0, :, ha * _D:(ha + 1) * _D]
            mmax = None
            msum = None
            for c in range(_L // _CH):
                kc = k2[c * _CH:(c + 1) * _CH, :]
                st = lax.dot_general(kc, q2, (((1,), (1,)), ((), ())),
                                     preferred_element_type=jnp.float32)
                pm = jnp.max(st + bias[c * _CH:(c + 1) * _CH, :],
                             axis=0, keepdims=True)
                ps = jnp.sum(st * cntf[c * _CH:(c + 1) * _CH, :],
                             axis=0, keepdims=True)
                mmax = pm if mmax is None else jnp.maximum(mmax, pm)
                msum = ps if msum is None else msum + ps
            mall[pl.ds(2 * i + ha, 1), :] = mmax - msum * (1.0 / _L)

    @pl.when(i == _NP)
    def _topk():
        lio = lax.broadcasted_iota(jnp.int32, (_BH, _L), 1)

        def topk_body(t, carry):
            m, s = carry
            mv = jnp.max(m, axis=1, keepdims=True)
            ii = jnp.min(jnp.where(m == mv, lio, _L), axis=1, keepdims=True)
            hit = lio == ii
            return jnp.where(hit, _NEG, m), jnp.where(hit, t, s)

        m0 = mall[...]
        s0 = jnp.full((_BH, _L), _L, jnp.int32)
        _, s = lax.fori_loop(0, _U, topk_body, (m0, s0))
        sall[...] = s

    @pl.when(i > _NP)
    def _stage2():
        p = i - (_NP + 1)
        halves = []
        for ha in range(2):
            q2 = q_ref[0, :, ha * _D:(ha + 1) * _D]
            k2 = k_ref[0, :, ha * _D:(ha + 1) * _D]
            v2 = v_ref[0, :, ha * _D:(ha + 1) * _D]
            srow = sall[pl.ds(2 * p + ha, 1), :]
            halves.append(_head_ctx(q2, k2, v2, srow))
        out_ref[0, :, :] = jnp.concatenate(halves, axis=1)


def kernel(query, key, value):
    cnt = _count_matrix()
    q3 = query.reshape(_B, _L, _HD)
    k3 = key.reshape(_B, _L, _HD)
    v3 = value.reshape(_B, _L, _HD)
    nhp = _H // 2

    def _pair_qk(i):
        j = jnp.where(i < _NP, i, jnp.maximum(i - (_NP + 1), 0))
        return (j // nhp, 0, j % nhp)

    def _pair_out(i):
        j = jnp.maximum(i - (_NP + 1), 0)
        return (j // nhp, 0, j % nhp)

    pair_qk = pl.BlockSpec((1, _L, 2 * _D), _pair_qk)
    pair_out = pl.BlockSpec((1, _L, 2 * _D), _pair_out)
    ctx = pl.pallas_call(
        _body,
        grid=(2 * _NP + 1,),
        in_specs=[
            pair_qk,
            pair_qk,
            pair_out,
            pl.BlockSpec((_L, _L), lambda i: (0, 0)),
        ],
        out_specs=pair_out,
        out_shape=jax.ShapeDtypeStruct((_B, _L, _HD), jnp.float32),
        scratch_shapes=[
            pltpu.VMEM((_BH, _L), jnp.float32),
            pltpu.VMEM((_BH, _L), jnp.int32),
            pltpu.VMEM((_L, _L), jnp.float32),
            pltpu.VMEM((_L, _L), jnp.float32),
        ],
    )(q3, k3, v3, cnt)
    return ctx.reshape(_B, _L, _H, _D)


# speedup vs baseline: 7.4904x; 1.0109x over previous
import functools

import numpy as np
import jax
import jax.numpy as jnp
from jax import lax
from jax.experimental import pallas as pl
from jax.experimental.pallas import tpu as pltpu

_B, _L, _H, _D = 2, 2048, 16, 64
_BH = _B * _H
_HD = _H * _D
_NP = _BH // 2
_U = 40
_UPAD = 48
_CH = 1024
_SCALE = 1.0 / 8.0
_NEG = -1e30


@functools.lru_cache(maxsize=1)
def _count_matrix_np() -> np.ndarray:
    with jax.ensure_compile_time_eval():
        idx = np.asarray(
            jax.random.randint(jax.random.key(42), (_L, _U), 0, _L))
    cnt = np.zeros((_L, _L), np.int8)
    np.add.at(cnt, (np.arange(_L)[:, None], idx), 1)
    return np.ascontiguousarray(cnt.T)


def _count_matrix():
    try:
        return jnp.asarray(_count_matrix_np())
    except Exception:
        idx = jax.random.randint(jax.random.key(42), (_L, _U), 0, _L)
        cnt = jnp.zeros((_L, _L), jnp.int8)
        return cnt.at[jnp.arange(_L)[:, None], idx].add(1).T


def _head_ctx(q2, k2, v2, srow):
    oh = (srow == lax.broadcasted_iota(jnp.int32, (_UPAD, 1), 0)
          ).astype(jnp.float32)
    qr = lax.dot_general(oh, q2, (((1,), (0,)), ((), ())),
                         preferred_element_type=jnp.float32)
    sc = lax.dot_general(qr, k2, (((1,), (1,)), ((), ())),
                         preferred_element_type=jnp.float32) * _SCALE
    sc = sc - jnp.max(sc, axis=1, keepdims=True)
    e = jnp.exp(sc)
    attn = e / jnp.sum(e, axis=1, keepdims=True)
    upd = lax.dot_general(attn, v2, (((1,), (0,)), ((), ())),
                          preferred_element_type=jnp.float32)
    vs = jnp.sum(v2, axis=0, keepdims=True)
    delta = lax.dot_general(oh, upd - vs, (((0,), (0,)), ((), ())),
                            preferred_element_type=jnp.float32)
    return vs + delta


def _body(q_ref, k_ref, v_ref, cnt_ref, out_ref, mall, sall, cntf, bias):
    i = pl.program_id(0)

    @pl.when(i == 0)
    def _expand_counts():
        for c in range(_L // _CH):
            cc = cnt_ref[c * _CH:(c + 1) * _CH, :].astype(jnp.float32)
            cntf[c * _CH:(c + 1) * _CH, :] = cc
            bias[c * _CH:(c + 1) * _CH, :] = jnp.where(cc > 0.0, 0.0, _NEG)

    @pl.when(i < _NP)
    def _stage1():
        for ha in range(2):
            q2 = q_ref[0, :, ha * _D:(ha + 1) * _D]
            k2 = k_ref[0, :, ha * _D:(ha + 1) * _D]
            mmax = None
            msum = None
            for c in range(_L // _CH):
                kc = k2[c * _CH:(c + 1) * _CH, :]
                st = lax.dot_general(kc, q2, (((1,), (1,)), ((), ())),
                                     preferred_element_type=jnp.float32)
                pm = jnp.max(st + bias[c * _CH:(c + 1) * _CH, :],
                             axis=0, keepdims=True)
                ps = jnp.sum(st * cntf[c * _CH:(c + 1) * _CH, :],
                             axis=0, keepdims=True)
                mmax = pm if mmax is None else jnp.maximum(mmax, pm)
                msum = ps if msum is None else msum + ps
            mall[pl.ds(2 * i + ha, 1), :] = mmax - msum * (1.0 / _L)

    @pl.when(i == _NP)
    def _topk():
        lio = lax.broadcasted_iota(jnp.int32, (_BH, _L), 1)

        def topk_body(t, carry):
            m, s = carry
            mv = jnp.max(m, axis=1, keepdims=True)
            ii = jnp.min(jnp.where(m == mv, lio, _L), axis=1, keepdims=True)
            hit = lio == ii
            return jnp.where(hit, _NEG, m), jnp.where(hit, t, s)

        m0 = mall[...]
        s0 = jnp.full((_BH, _L), _L, jnp.int32)
        _, s = lax.fori_loop(0, _U, topk_body, (m0, s0))
        sall[...] = s

    @pl.when(i > _NP)
    def _stage2():
        p = i - (_NP + 1)
        halves = []
        for ha in range(2):
            q2 = q_ref[0, :, ha * _D:(ha + 1) * _D]
            k2 = k_ref[0, :, ha * _D:(ha + 1) * _D]
            v2 = v_ref[0, :, ha * _D:(ha + 1) * _D]
            srow = sall[pl.ds(2 * p + ha, 1), :]
            halves.append(_head_ctx(q2, k2, v2, srow))
        out_ref[0, :, :] = jnp.concatenate(halves, axis=1)


def kernel(query, key, value):
    cnt = _count_matrix()
    q3 = query.reshape(_B, _L, _HD)
    k3 = key.reshape(_B, _L, _HD)
    v3 = value.reshape(_B, _L, _HD)
    nhp = _H // 2

    def _pair_qk(i):
        j = jnp.where(i < _NP, i, jnp.maximum(i - (_NP + 1), 0))
        return (j // nhp, 0, j % nhp)

    def _pair_out(i):
        j = jnp.maximum(i - (_NP + 1), 0)
        return (j // nhp, 0, j % nhp)

    pair_qk = pl.BlockSpec((1, _L, 2 * _D), _pair_qk)
    pair_out = pl.BlockSpec((1, _L, 2 * _D), _pair_out)
    ctx = pl.pallas_call(
        _body,
        grid=(2 * _NP + 1,),
        in_specs=[
            pair_qk,
            pair_qk,
            pair_out,
            pl.BlockSpec((_L, _L), lambda i: (0, 0)),
        ],
        out_specs=pair_out,
        out_shape=jax.ShapeDtypeStruct((_B, _L, _HD), jnp.float32),
        scratch_shapes=[
            pltpu.VMEM((_BH, _L), jnp.float32),
            pltpu.VMEM((_BH, _L), jnp.int32),
            pltpu.VMEM((_L, _L), jnp.float32),
            pltpu.VMEM((_L, _L), jnp.float32),
        ],
    )(q3, k3, v3, cnt)
    return ctx.reshape(_B, _L, _H, _D)
